# baseline (speedup 1.0000x reference)
import functools

import jax
import jax.numpy as jnp
from jax import lax
from jax.experimental import pallas as pl
from jax.experimental.pallas import tpu as pltpu
from jax.experimental.pallas import tpu_sc as plsc

NUM_FIELDS = 23
VOCAB = 100000
EMBED_DIM = 64
BATCH = 4096

NC = 2
NS = 16
NW = NC * NS
ROWS_PER_W = BATCH // NW
IDX_PER_W = ROWS_PER_W * NUM_FIELDS
CHUNK = 128
NCHUNK = IDX_PER_W // CHUNK
VEC = 16


def _sc_gather(flat_tables, flat_idx, offs):
    mesh = plsc.VectorSubcoreMesh(core_axis_name="c", subcore_axis_name="s")

    @functools.partial(
        pl.kernel,
        mesh=mesh,
        out_type=jax.ShapeDtypeStruct((BATCH * NUM_FIELDS, EMBED_DIM),
                                      jnp.float32),
        compiler_params=pltpu.CompilerParams(use_tc_tiling_on_sc=False),
        scratch_types=[
            pltpu.VMEM((IDX_PER_W,), jnp.int32),
            pltpu.VMEM((IDX_PER_W,), jnp.int32),
            pltpu.VMEM((CHUNK, EMBED_DIM), jnp.float32),
            pltpu.VMEM((CHUNK, EMBED_DIM), jnp.float32),
            pltpu.SemaphoreType.DMA,
            pltpu.SemaphoreType.DMA,
        ],
    )
    def k(table_hbm, idx_hbm, off_hbm, out_hbm,
          idx_v, off_v, rows0, rows1, sem0, sem1):
        wid = lax.axis_index("s") * NC + lax.axis_index("c")
        base = wid * IDX_PER_W
        pltpu.sync_copy(idx_hbm.at[pl.ds(base, IDX_PER_W)], idx_v)
        pltpu.sync_copy(off_hbm, off_v)

        @pl.loop(0, IDX_PER_W, step=VEC)
        def _(c):
            slc = pl.ds(c, VEC)
            idx_v.at[slc][...] = idx_v.at[slc][...] + off_v.at[slc][...]

        rows = (rows0, rows1)
        sems = (sem0, sem1)

        def start(j):
            return pltpu.async_copy(
                table_hbm.at[idx_v.at[pl.ds(j * CHUNK, CHUNK)]],
                rows[j % 2], sems[j % 2])

        cps = [None, None]
        cps[0] = start(0)
        for j in range(NCHUNK):
            if j + 1 < NCHUNK:
                cps[(j + 1) % 2] = start(j + 1)
            cps[j % 2].wait()
            pltpu.sync_copy(rows[j % 2],
                            out_hbm.at[pl.ds(base + j * CHUNK, CHUNK)])

    return k(flat_tables, flat_idx, offs)


def kernel(inputs, tables):
    flat_tables = tables.reshape(NUM_FIELDS * VOCAB, EMBED_DIM)
    flat_idx = inputs.reshape(-1)
    offs = jnp.tile(jnp.arange(NUM_FIELDS, dtype=jnp.int32) * VOCAB,
                    ROWS_PER_W)
    out = _sc_gather(flat_tables, flat_idx, offs)
    return out.reshape(BATCH, NUM_FIELDS * EMBED_DIM)


# speedup vs baseline: 3.9412x; 3.9412x over previous
import functools

import jax
import jax.numpy as jnp
from jax import lax
from jax.experimental import pallas as pl
from jax.experimental.pallas import tpu as pltpu
from jax.experimental.pallas import tpu_sc as plsc

NUM_FIELDS = 23
VOCAB = 100000
EMBED_DIM = 64
BATCH = 4096

NC = 2
NS = 16
NW = NC * NS
NPLANES = NUM_FIELDS * EMBED_DIM
PPW = NPLANES // NW
VEC = 16


def _sc_gather_planes(inputs_t, tables_t):
    mesh = plsc.VectorSubcoreMesh(core_axis_name="c", subcore_axis_name="s")

    @functools.partial(
        pl.kernel,
        mesh=mesh,
        out_type=jax.ShapeDtypeStruct((NPLANES, BATCH), jnp.float32),
        compiler_params=pltpu.CompilerParams(needs_layout_passes=False),
        scratch_types=[
            pltpu.VMEM((VOCAB,), jnp.float32),
            pltpu.VMEM((BATCH,), jnp.int32),
            pltpu.VMEM((BATCH,), jnp.float32),
        ],
    )
    def k(idx_hbm, tab_hbm, out_hbm, plane_v, idx_v, row_v):
        wid = lax.axis_index("s") * NC + lax.axis_index("c")
        p0 = wid * PPW

        @pl.loop(0, PPW)
        def _(j):
            p = p0 + j
            f = p // EMBED_DIM
            d = lax.rem(p, EMBED_DIM)
            pltpu.sync_copy(idx_hbm.at[f], idx_v)
            pltpu.sync_copy(tab_hbm.at[f, d], plane_v)

            @pl.loop(0, BATCH, step=VEC)
            def _(c):
                idx = idx_v.at[pl.ds(c, VEC)][...]
                row_v.at[pl.ds(c, VEC)][...] = plsc.load_gather(
                    plane_v, [idx])

            pltpu.sync_copy(row_v, out_hbm.at[p])

    return k(inputs_t, tables_t)


def kernel(inputs, tables):
    inputs_t = inputs.T
    tables_t = jnp.transpose(tables, (0, 2, 1))
    out_t = _sc_gather_planes(inputs_t, tables_t)
    return out_t.T
